# baseline (speedup 1.0000x reference)
import functools

import jax
import jax.numpy as jnp
from jax import lax
from jax.experimental import pallas as pl
from jax.experimental.pallas import tpu as pltpu
from jax.experimental.pallas import tpu_sc as plsc

DIM = 64
LANES = 16
NUM_CORES = 2
NUM_SUBCORES = 16
NUM_WORKERS = NUM_CORES * NUM_SUBCORES


def _sc_dot_kernel(n_total: int, chunk: int):
    per_w = n_total // NUM_WORKERS
    n_chunks = per_w // chunk
    assert n_chunks % 2 == 0
    mesh = plsc.VectorSubcoreMesh(core_axis_name="c", subcore_axis_name="s")

    @functools.partial(
        pl.kernel,
        out_type=jax.ShapeDtypeStruct((n_total,), jnp.float32),
        mesh=mesh,
        scratch_types=[
            pltpu.VMEM((chunk,), jnp.int32),
            pltpu.VMEM((chunk,), jnp.int32),
            pltpu.VMEM((chunk,), jnp.int32),
            pltpu.VMEM((chunk,), jnp.int32),
            pltpu.VMEM((chunk, DIM), jnp.bfloat16),
            pltpu.VMEM((chunk, DIM), jnp.bfloat16),
            pltpu.VMEM((chunk, DIM), jnp.bfloat16),
            pltpu.VMEM((chunk, DIM), jnp.bfloat16),
            pltpu.VMEM((chunk,), jnp.float32),
            pltpu.VMEM((chunk,), jnp.float32),
            pltpu.SemaphoreType.DMA,
            pltpu.SemaphoreType.DMA,
        ],
        compiler_params=pltpu.CompilerParams(
            use_tc_tiling_on_sc=False, needs_layout_passes=False
        ),
    )
    def kern(tgt_hbm, ctx_hbm, tt_hbm, ct_hbm, out_hbm,
             idx_t0, idx_t1, idx_c0, idx_c1,
             rows_t0, rows_t1, rows_c0, rows_c1,
             out_v0, out_v1, sem0, sem1):
        idx_t = (idx_t0, idx_t1)
        idx_c = (idx_c0, idx_c1)
        rows_t = (rows_t0, rows_t1)
        rows_c = (rows_c0, rows_c1)
        out_v = (out_v0, out_v1)
        sem = (sem0, sem1)
        wid = lax.axis_index("s") * NUM_CORES + lax.axis_index("c")
        wbase = wid * per_w

        def fire(slot, base):
            pltpu.sync_copy(tgt_hbm.at[pl.ds(base, chunk)], idx_t[slot])
            pltpu.sync_copy(ctx_hbm.at[pl.ds(base, chunk)], idx_c[slot])
            pltpu.async_copy(tt_hbm.at[idx_t[slot]], rows_t[slot], sem[slot])
            pltpu.async_copy(ct_hbm.at[idx_c[slot]], rows_c[slot], sem[slot])

        def drain(slot):
            pltpu.make_async_copy(
                tt_hbm.at[idx_t[slot]], rows_t[slot], sem[slot]).wait()
            pltpu.make_async_copy(
                ct_hbm.at[idx_c[slot]], rows_c[slot], sem[slot]).wait()

        def compute(slot, base):
            rt, rc, ov = rows_t[slot], rows_c[slot], out_v[slot]
            lanes = lax.iota(jnp.int32, LANES)

            def group_body(i, _):
                r0 = i * LANES
                out_acc = jnp.zeros((LANES,), jnp.float32)
                for rr in range(LANES):
                    r = r0 + rr
                    prods = []
                    for kk in range(DIM // (2 * LANES)):
                        tp = rt[r, pl.ds(kk * 2 * LANES, 2 * LANES)]
                        cp = rc[r, pl.ds(kk * 2 * LANES, 2 * LANES)]
                        ta, tb = plsc.unpack(
                            tp, format=plsc.PackFormat.INTERLEAVED)
                        ca, cb = plsc.unpack(
                            cp, format=plsc.PackFormat.INTERLEAVED)
                        prods.append(ta * ca)
                        prods.append(tb * cb)
                    s = (prods[0] + prods[1]) + (prods[2] + prods[3])
                    tot = jnp.sum(s)
                    out_acc = jnp.where(lanes == rr,
                                        jnp.full((LANES,), tot), out_acc)
                ov[pl.ds(r0, LANES)] = out_acc
                return 0

            lax.fori_loop(0, chunk // LANES, group_body, 0)
            pltpu.sync_copy(ov, out_hbm.at[pl.ds(base, chunk)])

        fire(0, wbase)

        def body(kk, _):
            c0 = wbase + (2 * kk) * chunk
            c1 = c0 + chunk
            fire(1, c1)
            drain(0)
            compute(0, c0)

            @pl.when(2 * kk + 2 < n_chunks)
            def _():
                fire(0, c1 + chunk)

            drain(1)
            compute(1, c1)
            return 0

        lax.fori_loop(0, n_chunks // 2, body, 0)

    return kern


def kernel(target, context, target_table, context_table):
    b, l = target.shape
    n_total = b * l
    tgt = target.reshape(n_total).astype(jnp.int32)
    ctx = context.reshape(n_total).astype(jnp.int32)
    tt = target_table.astype(jnp.bfloat16)
    ct = context_table.astype(jnp.bfloat16)
    sim = _sc_dot_kernel(n_total, chunk=256)(tgt, ctx, tt, ct)
    return sim.reshape(b, l)
